# baseline (speedup 1.0000x reference)
import functools
import math

import jax
import jax.numpy as jnp
from jax import lax
from jax.experimental import pallas as pl
from jax.experimental.pallas import tpu as pltpu
from jax.experimental.pallas import tpu_sc as plsc

N = 10000
RB = 400
NP = 10400
NBLK = NP // RB

E0 = 160000
EDGES = E0 + N
EPT = 11264
EPAD = 16 * EPT
K = 128
NRSL = 10496
ASTRIPE = NRSL // 16


def _prologue_body(z_ref, w_ref, as_ref, ad_ref, ht_ref, asrc_ref, adst_ref):
    j = pl.program_id(1)
    hj = jnp.dot(z_ref[...], w_ref[...], preferred_element_type=jnp.float32)
    ht_ref[0] = hj
    a_s = jnp.dot(hj, as_ref[0], preferred_element_type=jnp.float32)
    a_d = jnp.dot(hj, ad_ref[0], preferred_element_type=jnp.float32)

    @pl.when(j == 0)
    def _():
        asrc_ref[...] = a_s
        adst_ref[...] = a_d

    @pl.when(j != 0)
    def _():
        asrc_ref[...] += a_s
        adst_ref[...] += a_d


def tc_prologue(z, W, As8, Ad8):
    Din = z.shape[1]
    F = W.shape[1]
    H = As8.shape[2]
    nchunk = F // 128
    return pl.pallas_call(
        _prologue_body,
        grid=(NBLK, nchunk),
        in_specs=[
            pl.BlockSpec((RB, Din), lambda i, j: (i, 0)),
            pl.BlockSpec((Din, 128), lambda i, j: (0, j)),
            pl.BlockSpec((1, 128, H), lambda i, j: (j, 0, 0)),
            pl.BlockSpec((1, 128, H), lambda i, j: (j, 0, 0)),
        ],
        out_specs=[
            pl.BlockSpec((1, RB, 128), lambda i, j: (j, i, 0)),
            pl.BlockSpec((RB, H), lambda i, j: (i, 0)),
            pl.BlockSpec((RB, H), lambda i, j: (i, 0)),
        ],
        out_shape=[
            jax.ShapeDtypeStruct((nchunk, NP, 128), jnp.float32),
            jax.ShapeDtypeStruct((NP, H), jnp.float32),
            jax.ShapeDtypeStruct((NP, H), jnp.float32),
        ],
    )(z, W, As8, Ad8)


def _epi_concat_body(agg_ref, z_ref, wl_ref, b_ref, bl_ref, out_ref):
    skip = jnp.dot(z_ref[...], wl_ref[...], preferred_element_type=jnp.float32)
    v = agg_ref[0] + b_ref[0][None, :] + skip + bl_ref[0][None, :]
    out_ref[...] = jnp.where(v > 0, v, jnp.exp(v) - 1.0)


def tc_epilogue_concat(agg_t, z, Wl, b, bl):
    Din = z.shape[1]
    F = Wl.shape[1]
    nchunk = F // 128
    b2 = b.reshape(1, F)
    bl2 = bl.reshape(1, F)
    return pl.pallas_call(
        _epi_concat_body,
        grid=(NBLK, nchunk),
        in_specs=[
            pl.BlockSpec((1, RB, 128), lambda i, j: (j, i, 0)),
            pl.BlockSpec((RB, Din), lambda i, j: (i, 0)),
            pl.BlockSpec((Din, 128), lambda i, j: (0, j)),
            pl.BlockSpec((1, 128), lambda i, j: (0, j)),
            pl.BlockSpec((1, 128), lambda i, j: (0, j)),
        ],
        out_specs=pl.BlockSpec((RB, 128), lambda i, j: (i, j)),
        out_shape=jax.ShapeDtypeStruct((NP, F), jnp.float32),
    )(agg_t, z, Wl, b2, bl2)


def _epi_mean_body(agg_ref, z_ref, wl_ref, b_ref, bl_ref, out_ref):
    m = agg_ref[0]
    for k in range(1, 6):
        m = m + agg_ref[k]
    m = m * (1.0 / 6.0)
    skip = jnp.dot(z_ref[...], wl_ref[...], preferred_element_type=jnp.float32)
    out_ref[...] = m + b_ref[0][None, :] + skip + bl_ref[0][None, :]


def tc_epilogue_mean(agg_t, z, Wl, b, bl):
    Din = z.shape[1]
    b2 = b.reshape(1, 128)
    bl2 = bl.reshape(1, 128)
    return pl.pallas_call(
        _epi_mean_body,
        grid=(NBLK,),
        in_specs=[
            pl.BlockSpec((6, RB, 128), lambda i: (0, i, 0)),
            pl.BlockSpec((RB, Din), lambda i: (i, 0)),
            pl.BlockSpec((Din, 128), lambda i: (0, 0)),
            pl.BlockSpec((1, 128), lambda i: (0, 0)),
            pl.BlockSpec((1, 128), lambda i: (0, 0)),
        ],
        out_specs=pl.BlockSpec((RB, 128), lambda i: (i, 0)),
        out_shape=jax.ShapeDtypeStruct((NP, 128), jnp.float32),
    )(agg_t, z, Wl, b2, bl2)


_SC_MESH = plsc.VectorSubcoreMesh(core_axis_name="c", subcore_axis_name="s",
                                  num_cores=2, num_subcores=16)


def _zero_1d(ref, n):
    zeros = jnp.zeros((16,), jnp.float32)

    def body(i, _):
        ref[pl.ds(i * 16, 16)] = zeros
        return 0

    lax.fori_loop(0, n // 16, body, 0)


def make_sc_coef(H):
    Hc = H // 2
    KC = 512
    NB1 = EPT // KC

    scratch = []
    scratch += [pltpu.VMEM((NP,), jnp.float32) for _ in range(2 * Hc)]
    scratch += [pltpu.VMEM((NRSL,), jnp.float32) for _ in range(Hc)]
    scratch += [
        pltpu.VMEM((KC,), jnp.int32),
        pltpu.VMEM((KC,), jnp.int32),
        pltpu.VMEM((KC,), jnp.float32),
        pltpu.VMEM((ASTRIPE,), jnp.float32),
    ]
    scratch += [pltpu.VMEM((ASTRIPE,), jnp.float32) for _ in range(16)]
    scratch += [pltpu.SemaphoreType.DMA]

    @functools.partial(
        pl.kernel,
        out_type=(
            jax.ShapeDtypeStruct((H, EPAD), jnp.float32),
            jax.ShapeDtypeStruct((32 * Hc * NRSL,), jnp.float32),
            jax.ShapeDtypeStruct((H * NRSL,), jnp.float32),
        ),
        mesh=_SC_MESH,
        compiler_params=pltpu.CompilerParams(needs_layout_passes=False),
        scratch_types=scratch,
    )
    def sc_coef(asrcT, adstT, srcp, dstp, coef_out, part, sfin, *refs):
        as_pl = refs[0:Hc]
        ad_pl = refs[Hc:2 * Hc]
        s_pl = refs[2 * Hc:3 * Hc]
        sbuf, dbuf, cbuf, cacc = refs[3 * Hc:3 * Hc + 4]
        ctmp = refs[3 * Hc + 4:3 * Hc + 20]
        sem = refs[3 * Hc + 20]

        core = lax.axis_index("c")
        tile = lax.axis_index("s")
        base_e = tile * EPT
        soff = tile * ASTRIPE
        wid = core * 16 + tile

        for hh in range(Hc):
            pltpu.sync_copy(asrcT.at[core * Hc + hh], as_pl[hh])
            pltpu.sync_copy(adstT.at[core * Hc + hh], ad_pl[hh])
            _zero_1d(s_pl[hh], NRSL)

        def batch_a(b, _):
            off = base_e + b * KC
            pltpu.sync_copy(srcp.at[pl.ds(off, KC)], sbuf)
            pltpu.sync_copy(dstp.at[pl.ds(off, KC)], dbuf)

            def vec(v, _):
                sl = pl.ds(v * 16, 16)
                sv = sbuf[sl]
                dv = dbuf[sl]
                for hh in range(Hc):
                    av = (plsc.load_gather(as_pl[hh], [sv])
                          + plsc.load_gather(ad_pl[hh], [dv]))
                    av = jnp.where(av >= 0, av, 0.2 * av)
                    plsc.addupdate_scatter(s_pl[hh], [dv], jnp.exp(av))
                return 0

            lax.fori_loop(0, KC // 16, vec, 0)
            return 0

        lax.fori_loop(0, NB1, batch_a, 0)
        for hh in range(Hc):
            pltpu.sync_copy(s_pl[hh],
                            part.at[pl.ds((wid * Hc + hh) * NRSL, NRSL)])

        plsc.subcore_barrier()

        for hh in range(Hc):
            for p in range(16):
                pltpu.async_copy(
                    part.at[pl.ds(((core * 16 + p) * Hc + hh) * NRSL + soff,
                                  ASTRIPE)],
                    ctmp[p], sem)
            for p in range(16):
                pltpu.make_async_copy(
                    part.at[pl.ds(((core * 16 + p) * Hc + hh) * NRSL + soff,
                                  ASTRIPE)],
                    ctmp[p], sem).wait()
            _zero_1d(cacc, ASTRIPE)

            def vadd(v, _):
                sl = pl.ds(v * 16, 16)
                acc16 = cacc[sl]
                for p in range(16):
                    acc16 = acc16 + ctmp[p][sl]
                cacc[sl] = acc16
                return 0

            lax.fori_loop(0, ASTRIPE // 16, vadd, 0)
            pltpu.sync_copy(
                cacc, sfin.at[pl.ds((core * Hc + hh) * NRSL + soff, ASTRIPE)])

        plsc.subcore_barrier()

        for hh in range(Hc):
            pltpu.sync_copy(sfin.at[pl.ds((core * Hc + hh) * NRSL, NRSL)],
                            s_pl[hh])

        def batch_b(b, _):
            off = base_e + b * KC
            pltpu.sync_copy(srcp.at[pl.ds(off, KC)], sbuf)
            pltpu.sync_copy(dstp.at[pl.ds(off, KC)], dbuf)
            for hh in range(Hc):
                def vec(v, _):
                    sl = pl.ds(v * 16, 16)
                    sv = sbuf[sl]
                    dv = dbuf[sl]
                    av = (plsc.load_gather(as_pl[hh], [sv])
                          + plsc.load_gather(ad_pl[hh], [dv]))
                    av = jnp.where(av >= 0, av, 0.2 * av)
                    ev = jnp.exp(av)
                    sval = plsc.load_gather(s_pl[hh], [dv])
                    cbuf[sl] = ev / (sval + 1e-16)
                    return 0

                lax.fori_loop(0, KC // 16, vec, 0)
                pltpu.sync_copy(cbuf,
                                coef_out.at[core * Hc + hh, pl.ds(off, KC)])
            return 0

        lax.fori_loop(0, NB1, batch_b, 0)

    return sc_coef


NRACC = NRSL


def make_sc_aggregate(nchunk, H):
    CPC = nchunk // 2
    sub = nchunk // H
    SBR = 8
    NSUP = EPT // (SBR * 128)

    slab_i = pltpu.VMEM((SBR, 128), jnp.int32)

    @functools.partial(
        pl.kernel,
        out_type=jax.ShapeDtypeStruct((nchunk, NRACC, 128), jnp.float32),
        mesh=_SC_MESH,
        compiler_params=pltpu.CompilerParams(needs_layout_passes=False),
        scratch_types=[
            slab_i,
            slab_i,
            pltpu.VMEM((SBR, 128), jnp.float32),
            slab_i,
            pltpu.VMEM((128, 128), jnp.float32),
            pltpu.VMEM((128, 128), jnp.float32),
            pltpu.SemaphoreType.DMA,
            pltpu.SemaphoreType.DMA,
            pltpu.SemaphoreType.DMA,
            pltpu.SemaphoreType.DMA,
            pltpu.VMEM_SHARED((NRACC, 128), jnp.float32),
        ],
    )
    def sc_agg(ht, coef3, src2, dst2, agg_out,
               sslab, dslab, cslab, gslab, rw0, rw1,
               semg0, semg1, sems0, sems1, acc):
        core = lax.axis_index("c")
        tile = lax.axis_index("s")
        soff = tile * ASTRIPE
        rbase = tile * (EPT // 128)
        zeros = jnp.zeros((16,), jnp.float32)
        rws = (rw0, rw1)
        semg = (semg0, semg1)
        sems = (sems0, sems1)

        def zero_stripe():
            def zrow(r, _):
                for q in range(8):
                    rw0[r, pl.ds(q * 16, 16)] = zeros
                return 0

            lax.fori_loop(0, 128, zrow, 0)
            for i in range(ASTRIPE // 128):
                pltpu.sync_copy(rw0, acc.at[pl.ds(soff + i * 128, 128)])
            rem = ASTRIPE % 128
            pltpu.sync_copy(rw0.at[pl.ds(0, rem)],
                            acc.at[pl.ds(soff + ASTRIPE - rem, rem)])

        zero_stripe()

        def chunk_body(cc, _):
            c = core * CPC + cc
            head = c // sub
            cbase = c * NP

            plsc.subcore_barrier()

            def start_g(buf, j):
                pltpu.async_copy(ht.at[gslab.at[j]], rws[buf], semg[buf])

            def wait_g(buf, j):
                pltpu.make_async_copy(ht.at[gslab.at[j]], rws[buf],
                                      semg[buf]).wait()

            def start_s(buf, j):
                pltpu.async_copy(rws[buf], acc.at[dslab.at[j]], sems[buf],
                                 add=True)

            def wait_s(buf, j):
                pltpu.make_async_copy(rws[buf], acc.at[dslab.at[j]],
                                      sems[buf]).wait()

            def mul(buf, j):
                rw = rws[buf]

                def mv(v, _):
                    cv = cslab[j, pl.ds(v * 16, 16)]
                    for lane in range(16):
                        e = v * 16 + lane
                        cf = cv[lane]
                        for q in range(8):
                            sl = pl.ds(q * 16, 16)
                            rw[e, sl] = rw[e, sl] * cf
                    return 0

                lax.fori_loop(0, 8, mv, 0)

            def super_body(sp, _):
                r0 = rbase + sp * SBR
                pltpu.sync_copy(src2.at[pl.ds(r0, SBR)], sslab)
                pltpu.sync_copy(dst2.at[pl.ds(r0, SBR)], dslab)
                pltpu.sync_copy(coef3.at[head, pl.ds(r0, SBR)], cslab)

                def gv(r, _):
                    for q in range(8):
                        sl = pl.ds(q * 16, 16)
                        gslab[r, sl] = sslab[r, sl] + cbase
                    return 0

                lax.fori_loop(0, SBR, gv, 0)

                start_g(0, 0)

                def pair(g, _):
                    j0 = 2 * g
                    j1 = j0 + 1
                    start_g(1, j1)
                    wait_g(0, j0)
                    mul(0, j0)
                    start_s(0, j0)
                    wait_g(1, j1)
                    mul(1, j1)
                    start_s(1, j1)
                    wait_s(0, j0)

                    @pl.when(g < SBR // 2 - 1)
                    def _():
                        start_g(0, j0 + 2)

                    wait_s(1, j1)
                    return 0

                lax.fori_loop(0, SBR // 2, pair, 0)
                return 0

            lax.fori_loop(0, NSUP, super_body, 0)

            plsc.subcore_barrier()

            pltpu.sync_copy(acc.at[pl.ds(soff, ASTRIPE)],
                            agg_out.at[c, pl.ds(soff, ASTRIPE)])
            zero_stripe()
            return 0

        lax.fori_loop(0, CPC, chunk_body, 0)

    return sc_agg


def _att_blockdiag(att, F):
    H, C = att.shape
    nchunk = F // 128
    sub = C // 128
    A = jnp.zeros((nchunk, 128, H), jnp.float32)
    for c in range(nchunk):
        hd = c // sub
        block = att[hd, (c % sub) * 128:(c % sub + 1) * 128]
        A = A.at[c, :, hd].set(block)
    return A


def _gat_layer(z, src, dst, W, att_s, att_d, b, Wl, bl, heads, concat):
    F = W.shape[1]
    nchunk = F // 128
    As8 = _att_blockdiag(att_s, F)
    Ad8 = _att_blockdiag(att_d, F)
    h_t, asrc, adst = tc_prologue(z, W, As8, Ad8)
    coef_p, _, _ = make_sc_coef(heads)(asrc.T, adst.T, src, dst)
    ht_flat = h_t.reshape(nchunk * NP, 128)
    coef3 = coef_p.reshape(heads, EPAD // 128, 128)
    src2 = src.reshape(EPAD // 128, 128)
    dst2 = dst.reshape(EPAD // 128, 128)
    agg_t = make_sc_aggregate(nchunk, heads)(ht_flat, coef3, src2, dst2)
    if concat:
        return tc_epilogue_concat(agg_t, z, Wl, b, bl)
    return tc_epilogue_mean(agg_t, z, Wl, b, bl)


def kernel(x, edge_index, W1, as1, ad1, b1, Wl1, bl1, W2, as2, ad2, b2, Wl2,
           bl2, W3, as3, ad3, b3, Wl3, bl3):
    loop = jnp.arange(N, dtype=edge_index.dtype)
    ei = jnp.concatenate([edge_index, jnp.stack([loop, loop])], axis=1)
    src = jnp.full((EPAD,), N, jnp.int32).at[:EDGES].set(ei[0].astype(jnp.int32))
    dst = jnp.full((EPAD,), N, jnp.int32).at[:EDGES].set(ei[1].astype(jnp.int32))

    xp = jnp.zeros((NP, x.shape[1]), jnp.float32).at[:N].set(x)
    h = _gat_layer(xp, src, dst, W1, as1, ad1, b1, Wl1, bl1, 4, True)
    h = _gat_layer(h, src, dst, W2, as2, ad2, b2, Wl2, bl2, 4, True)
    out = _gat_layer(h, src, dst, W3, as3, ad3, b3, Wl3, bl3, 6, False)
    return out[:N]


# speedup vs baseline: 2.0559x; 2.0559x over previous
import functools
import math

import jax
import jax.numpy as jnp
from jax import lax
from jax.experimental import pallas as pl
from jax.experimental.pallas import tpu as pltpu
from jax.experimental.pallas import tpu_sc as plsc

N = 10000
RB = 400
NP = 10400
NBLK = NP // RB

E0 = 160000
EDGES = E0 + N
EPT = 10752
EPAD = 16 * EPT
K = 128
NRSL = 10496
ASTRIPE = NRSL // 16


def _prologue_body(z_ref, w_ref, as_ref, ad_ref, ht_ref, asrc_ref, adst_ref):
    j = pl.program_id(1)
    hj = jnp.dot(z_ref[...], w_ref[...], preferred_element_type=jnp.float32)
    ht_ref[0] = hj
    a_s = jnp.dot(hj, as_ref[0], preferred_element_type=jnp.float32)
    a_d = jnp.dot(hj, ad_ref[0], preferred_element_type=jnp.float32)

    @pl.when(j == 0)
    def _():
        asrc_ref[...] = a_s
        adst_ref[...] = a_d

    @pl.when(j != 0)
    def _():
        asrc_ref[...] += a_s
        adst_ref[...] += a_d


def tc_prologue(z, W, As8, Ad8):
    Din = z.shape[1]
    F = W.shape[1]
    H = As8.shape[2]
    nchunk = F // 128
    return pl.pallas_call(
        _prologue_body,
        grid=(NBLK, nchunk),
        in_specs=[
            pl.BlockSpec((RB, Din), lambda i, j: (i, 0)),
            pl.BlockSpec((Din, 128), lambda i, j: (0, j)),
            pl.BlockSpec((1, 128, H), lambda i, j: (j, 0, 0)),
            pl.BlockSpec((1, 128, H), lambda i, j: (j, 0, 0)),
        ],
        out_specs=[
            pl.BlockSpec((1, RB, 128), lambda i, j: (j, i, 0)),
            pl.BlockSpec((RB, H), lambda i, j: (i, 0)),
            pl.BlockSpec((RB, H), lambda i, j: (i, 0)),
        ],
        out_shape=[
            jax.ShapeDtypeStruct((nchunk, NP, 128), jnp.float32),
            jax.ShapeDtypeStruct((NP, H), jnp.float32),
            jax.ShapeDtypeStruct((NP, H), jnp.float32),
        ],
    )(z, W, As8, Ad8)


def _epi_concat_body(agg_ref, z_ref, wl_ref, b_ref, bl_ref, out_ref):
    skip = jnp.dot(z_ref[...], wl_ref[...], preferred_element_type=jnp.float32)
    v = agg_ref[0] + b_ref[0][None, :] + skip + bl_ref[0][None, :]
    out_ref[...] = jnp.where(v > 0, v, jnp.exp(v) - 1.0)


def tc_epilogue_concat(agg_t, z, Wl, b, bl):
    Din = z.shape[1]
    F = Wl.shape[1]
    nchunk = F // 128
    b2 = b.reshape(1, F)
    bl2 = bl.reshape(1, F)
    return pl.pallas_call(
        _epi_concat_body,
        grid=(NBLK, nchunk),
        in_specs=[
            pl.BlockSpec((1, RB, 128), lambda i, j: (j, i, 0)),
            pl.BlockSpec((RB, Din), lambda i, j: (i, 0)),
            pl.BlockSpec((Din, 128), lambda i, j: (0, j)),
            pl.BlockSpec((1, 128), lambda i, j: (0, j)),
            pl.BlockSpec((1, 128), lambda i, j: (0, j)),
        ],
        out_specs=pl.BlockSpec((RB, 128), lambda i, j: (i, j)),
        out_shape=jax.ShapeDtypeStruct((NP, F), jnp.float32),
    )(agg_t, z, Wl, b2, bl2)


def _epi_mean_body(agg_ref, z_ref, wl_ref, b_ref, bl_ref, out_ref):
    m = agg_ref[0]
    for k in range(1, 6):
        m = m + agg_ref[k]
    m = m * (1.0 / 6.0)
    skip = jnp.dot(z_ref[...], wl_ref[...], preferred_element_type=jnp.float32)
    out_ref[...] = m + b_ref[0][None, :] + skip + bl_ref[0][None, :]


def tc_epilogue_mean(agg_t, z, Wl, b, bl):
    Din = z.shape[1]
    b2 = b.reshape(1, 128)
    bl2 = bl.reshape(1, 128)
    return pl.pallas_call(
        _epi_mean_body,
        grid=(NBLK,),
        in_specs=[
            pl.BlockSpec((6, RB, 128), lambda i: (0, i, 0)),
            pl.BlockSpec((RB, Din), lambda i: (i, 0)),
            pl.BlockSpec((Din, 128), lambda i: (0, 0)),
            pl.BlockSpec((1, 128), lambda i: (0, 0)),
            pl.BlockSpec((1, 128), lambda i: (0, 0)),
        ],
        out_specs=pl.BlockSpec((RB, 128), lambda i: (i, 0)),
        out_shape=jax.ShapeDtypeStruct((NP, 128), jnp.float32),
    )(agg_t, z, Wl, b2, bl2)


_SC_MESH = plsc.VectorSubcoreMesh(core_axis_name="c", subcore_axis_name="s",
                                  num_cores=2, num_subcores=16)


def _zero_1d(ref, n):
    zeros = jnp.zeros((16,), jnp.float32)

    def body(i, _):
        ref[pl.ds(i * 16, 16)] = zeros
        return 0

    lax.fori_loop(0, n // 16, body, 0)


def make_sc_coef(H):
    Hc = H // 2
    KC = 512
    NB1 = EPT // KC

    scratch = []
    scratch += [pltpu.VMEM((NP,), jnp.float32) for _ in range(2 * Hc)]
    scratch += [pltpu.VMEM((NRSL,), jnp.float32) for _ in range(Hc)]
    scratch += [
        pltpu.VMEM((KC,), jnp.int32),
        pltpu.VMEM((KC,), jnp.int32),
        pltpu.VMEM((KC,), jnp.float32),
        pltpu.VMEM((ASTRIPE,), jnp.float32),
    ]
    scratch += [pltpu.VMEM((ASTRIPE,), jnp.float32) for _ in range(16)]
    scratch += [pltpu.SemaphoreType.DMA]

    @functools.partial(
        pl.kernel,
        out_type=(
            jax.ShapeDtypeStruct((H, EPAD), jnp.float32),
            jax.ShapeDtypeStruct((32 * Hc * NRSL,), jnp.float32),
            jax.ShapeDtypeStruct((H * NRSL,), jnp.float32),
        ),
        mesh=_SC_MESH,
        compiler_params=pltpu.CompilerParams(needs_layout_passes=False),
        scratch_types=scratch,
    )
    def sc_coef(asrcT, adstT, srcp, dstp, coef_out, part, sfin, *refs):
        as_pl = refs[0:Hc]
        ad_pl = refs[Hc:2 * Hc]
        s_pl = refs[2 * Hc:3 * Hc]
        sbuf, dbuf, cbuf, cacc = refs[3 * Hc:3 * Hc + 4]
        ctmp = refs[3 * Hc + 4:3 * Hc + 20]
        sem = refs[3 * Hc + 20]

        core = lax.axis_index("c")
        tile = lax.axis_index("s")
        base_e = tile * EPT
        soff = tile * ASTRIPE
        wid = core * 16 + tile

        for hh in range(Hc):
            pltpu.sync_copy(asrcT.at[core * Hc + hh], as_pl[hh])
            pltpu.sync_copy(adstT.at[core * Hc + hh], ad_pl[hh])
            _zero_1d(s_pl[hh], NRSL)

        def batch_a(b, _):
            off = base_e + b * KC
            pltpu.sync_copy(srcp.at[pl.ds(off, KC)], sbuf)
            pltpu.sync_copy(dstp.at[pl.ds(off, KC)], dbuf)

            def vec(v, _):
                sl = pl.ds(v * 16, 16)
                sv = sbuf[sl]
                dv = dbuf[sl]
                for hh in range(Hc):
                    av = (plsc.load_gather(as_pl[hh], [sv])
                          + plsc.load_gather(ad_pl[hh], [dv]))
                    av = jnp.where(av >= 0, av, 0.2 * av)
                    plsc.addupdate_scatter(s_pl[hh], [dv], jnp.exp(av))
                return 0

            lax.fori_loop(0, KC // 16, vec, 0)
            return 0

        lax.fori_loop(0, NB1, batch_a, 0)
        for hh in range(Hc):
            pltpu.sync_copy(s_pl[hh],
                            part.at[pl.ds((wid * Hc + hh) * NRSL, NRSL)])

        plsc.subcore_barrier()

        for hh in range(Hc):
            for p in range(16):
                pltpu.async_copy(
                    part.at[pl.ds(((core * 16 + p) * Hc + hh) * NRSL + soff,
                                  ASTRIPE)],
                    ctmp[p], sem)
            for p in range(16):
                pltpu.make_async_copy(
                    part.at[pl.ds(((core * 16 + p) * Hc + hh) * NRSL + soff,
                                  ASTRIPE)],
                    ctmp[p], sem).wait()
            _zero_1d(cacc, ASTRIPE)

            def vadd(v, _):
                sl = pl.ds(v * 16, 16)
                acc16 = cacc[sl]
                for p in range(16):
                    acc16 = acc16 + ctmp[p][sl]
                cacc[sl] = acc16
                return 0

            lax.fori_loop(0, ASTRIPE // 16, vadd, 0)
            pltpu.sync_copy(
                cacc, sfin.at[pl.ds((core * Hc + hh) * NRSL + soff, ASTRIPE)])

        plsc.subcore_barrier()

        for hh in range(Hc):
            pltpu.sync_copy(sfin.at[pl.ds((core * Hc + hh) * NRSL, NRSL)],
                            s_pl[hh])

        def batch_b(b, _):
            off = base_e + b * KC
            pltpu.sync_copy(srcp.at[pl.ds(off, KC)], sbuf)
            pltpu.sync_copy(dstp.at[pl.ds(off, KC)], dbuf)
            for hh in range(Hc):
                def vec(v, _):
                    sl = pl.ds(v * 16, 16)
                    sv = sbuf[sl]
                    dv = dbuf[sl]
                    av = (plsc.load_gather(as_pl[hh], [sv])
                          + plsc.load_gather(ad_pl[hh], [dv]))
                    av = jnp.where(av >= 0, av, 0.2 * av)
                    ev = jnp.exp(av)
                    sval = plsc.load_gather(s_pl[hh], [dv])
                    cbuf[sl] = ev / (sval + 1e-16)
                    return 0

                lax.fori_loop(0, KC // 16, vec, 0)
                pltpu.sync_copy(cbuf,
                                coef_out.at[core * Hc + hh, pl.ds(off, KC)])
            return 0

        lax.fori_loop(0, NB1, batch_b, 0)

    return sc_coef


NRACC = NRSL


def make_sc_aggregate(nchunk, H):
    CPC = nchunk // 2
    sub = nchunk // H
    KA = 128
    NBA = EPT // KA
    NB2 = NBA // 2

    idx_t = pltpu.VMEM((1, 128), jnp.int32)
    cf_t = pltpu.VMEM((1, 128), jnp.float32)

    @functools.partial(
        pl.kernel,
        out_type=jax.ShapeDtypeStruct((nchunk, NRACC, 128), jnp.float32),
        mesh=_SC_MESH,
        compiler_params=pltpu.CompilerParams(needs_layout_passes=False),
        scratch_types=[
            idx_t, idx_t, cf_t, idx_t, idx_t,
            idx_t, idx_t, cf_t, idx_t, idx_t,
            pltpu.VMEM((KA, 128), jnp.float32),
            pltpu.VMEM((KA, 128), jnp.float32),
            pltpu.SemaphoreType.DMA,
            pltpu.SemaphoreType.DMA,
            pltpu.SemaphoreType.DMA,
            pltpu.SemaphoreType.DMA,
            pltpu.VMEM_SHARED((NRACC, 128), jnp.float32),
        ],
    )
    def sc_agg(ht, coef, srcp, dstp, agg_out,
               sb0, db0, cb0, gb0, x0, sb1, db1, cb1, gb1, x1, rw0, rw1,
               semg0, semg1, sems0, sems1, acc):
        core = lax.axis_index("c")
        tile = lax.axis_index("s")
        base_e = tile * EPT
        soff = tile * ASTRIPE
        zeros = jnp.zeros((16,), jnp.float32)
        bufs = ((sb0, db0, cb0, gb0, x0, rw0, semg0, sems0),
                (sb1, db1, cb1, gb1, x1, rw1, semg1, sems1))

        def zero_stripe():
            def zrow(r, _):
                for j in range(8):
                    rw0[r, pl.ds(j * 16, 16)] = zeros
                return 0

            lax.fori_loop(0, KA, zrow, 0)
            for i in range(ASTRIPE // KA):
                pltpu.sync_copy(rw0, acc.at[pl.ds(soff + i * KA, KA)])
            rem = ASTRIPE % KA
            pltpu.sync_copy(rw0.at[pl.ds(0, rem)],
                            acc.at[pl.ds(soff + ASTRIPE - rem, rem)])

        zero_stripe()

        for cc in range(CPC):
            c = core * CPC + cc
            head = c // sub
            cbase = c * NP

            plsc.subcore_barrier()

            def load_idx(which, b):
                sb, db, cb, gb = bufs[which][:4]
                off = base_e + b * KA
                pltpu.sync_copy(srcp.at[pl.ds(off, 128)], sb.at[0])
                pltpu.sync_copy(dstp.at[pl.ds(off, 128)], db.at[0])
                pltpu.sync_copy(coef.at[head, pl.ds(off, 128)], cb.at[0])

                def gv(v, _):
                    sl = pl.ds(v * 16, 16)
                    gb[0, sl] = sb[0, sl] + cbase
                    return 0

                lax.fori_loop(0, 8, gv, 0)

            def start_g(which):
                gb, rw, semg = bufs[which][3], bufs[which][5], bufs[which][6]
                pltpu.async_copy(ht.at[gb.at[0]], rw, semg)

            def wait_g(which):
                gb, rw, semg = bufs[which][3], bufs[which][5], bufs[which][6]
                pltpu.make_async_copy(ht.at[gb.at[0]], rw, semg).wait()

            def start_s(which):
                xb, rw, sems = bufs[which][4], bufs[which][5], bufs[which][7]
                pltpu.async_copy(rw, acc.at[xb.at[0]], sems, add=True)

            def wait_s(which):
                xb, rw, sems = bufs[which][4], bufs[which][5], bufs[which][7]
                pltpu.make_async_copy(rw, acc.at[xb.at[0]], sems).wait()

            def process(which):
                db, cb, xb, rw = (bufs[which][1], bufs[which][2],
                                  bufs[which][4], bufs[which][5])

                def mul(v, _):
                    cv = cb[0, pl.ds(v * 16, 16)]
                    for lane in range(16):
                        e = v * 16 + lane
                        cf = cv[lane]
                        for q in range(8):
                            sl = pl.ds(q * 16, 16)
                            rw[e, sl] = rw[e, sl] * cf
                    return 0

                lax.fori_loop(0, 8, mul, 0)

                def cpy(v, _):
                    sl = pl.ds(v * 16, 16)
                    xb[0, sl] = db[0, sl]
                    return 0

                lax.fori_loop(0, 8, cpy, 0)
                start_s(which)

            load_idx(0, 0)
            start_g(0)

            def pair(i, _):
                load_idx(1, 2 * i + 1)

                @pl.when(i > 0)
                def _():
                    wait_s(1)

                start_g(1)
                wait_g(0)
                process(0)

                @pl.when(i < NB2 - 1)
                def _():
                    load_idx(0, 2 * i + 2)
                    wait_s(0)
                    start_g(0)

                wait_g(1)
                process(1)
                return 0

            lax.fori_loop(0, NB2, pair, 0)
            wait_s(0)
            wait_s(1)

            plsc.subcore_barrier()

            pltpu.sync_copy(acc.at[pl.ds(soff, ASTRIPE)],
                            agg_out.at[c, pl.ds(soff, ASTRIPE)])
            zero_stripe()

    return sc_agg


def _att_blockdiag(att, F):
    H, C = att.shape
    nchunk = F // 128
    sub = C // 128
    A = jnp.zeros((nchunk, 128, H), jnp.float32)
    for c in range(nchunk):
        hd = c // sub
        block = att[hd, (c % sub) * 128:(c % sub + 1) * 128]
        A = A.at[c, :, hd].set(block)
    return A


def _gat_layer(z, src, dst, W, att_s, att_d, b, Wl, bl, heads, concat):
    F = W.shape[1]
    nchunk = F // 128
    As8 = _att_blockdiag(att_s, F)
    Ad8 = _att_blockdiag(att_d, F)
    h_t, asrc, adst = tc_prologue(z, W, As8, Ad8)
    coef_p, _, _ = make_sc_coef(heads)(asrc.T, adst.T, src, dst)
    ht_flat = h_t.reshape(nchunk * NP, 128)
    agg_t = make_sc_aggregate(nchunk, heads)(ht_flat, coef_p, src, dst)
    if concat:
        return tc_epilogue_concat(agg_t, z, Wl, b, bl)
    return tc_epilogue_mean(agg_t, z, Wl, b, bl)


def kernel(x, edge_index, W1, as1, ad1, b1, Wl1, bl1, W2, as2, ad2, b2, Wl2,
           bl2, W3, as3, ad3, b3, Wl3, bl3):
    loop = jnp.arange(N, dtype=edge_index.dtype)
    ei = jnp.concatenate([edge_index, jnp.stack([loop, loop])], axis=1)
    src = jnp.full((EPAD,), N, jnp.int32).at[:EDGES].set(ei[0].astype(jnp.int32))
    dst = jnp.full((EPAD,), N, jnp.int32).at[:EDGES].set(ei[1].astype(jnp.int32))

    xp = jnp.zeros((NP, x.shape[1]), jnp.float32).at[:N].set(x)
    h = _gat_layer(xp, src, dst, W1, as1, ad1, b1, Wl1, bl1, 4, True)
    h = _gat_layer(h, src, dst, W2, as2, ad2, b2, Wl2, bl2, 4, True)
    out = _gat_layer(h, src, dst, W3, as3, ad3, b3, Wl3, bl3, 6, False)
    return out[:N]


# speedup vs baseline: 2.2205x; 1.0801x over previous
import functools
import math

import jax
import jax.numpy as jnp
from jax import lax
from jax.experimental import pallas as pl
from jax.experimental.pallas import tpu as pltpu
from jax.experimental.pallas import tpu_sc as plsc

N = 10000
RB = 400
NP = 10400
NBLK = NP // RB

E0 = 160000
EDGES = E0 + N
EPT = 10752
EPAD = 16 * EPT
K = 128
NRSL = 10496
ASTRIPE = NRSL // 16


def _prologue_body(z_ref, w_ref, as_ref, ad_ref, ht_ref, asrc_ref, adst_ref):
    j = pl.program_id(1)
    hj = jnp.dot(z_ref[...], w_ref[...], preferred_element_type=jnp.float32)
    ht_ref[0] = hj
    a_s = jnp.dot(hj, as_ref[0], preferred_element_type=jnp.float32)
    a_d = jnp.dot(hj, ad_ref[0], preferred_element_type=jnp.float32)

    @pl.when(j == 0)
    def _():
        asrc_ref[...] = a_s
        adst_ref[...] = a_d

    @pl.when(j != 0)
    def _():
        asrc_ref[...] += a_s
        adst_ref[...] += a_d


def tc_prologue(z, W, As8, Ad8):
    Din = z.shape[1]
    F = W.shape[1]
    H = As8.shape[2]
    nchunk = F // 128
    return pl.pallas_call(
        _prologue_body,
        grid=(NBLK, nchunk),
        in_specs=[
            pl.BlockSpec((RB, Din), lambda i, j: (i, 0)),
            pl.BlockSpec((Din, 128), lambda i, j: (0, j)),
            pl.BlockSpec((1, 128, H), lambda i, j: (j, 0, 0)),
            pl.BlockSpec((1, 128, H), lambda i, j: (j, 0, 0)),
        ],
        out_specs=[
            pl.BlockSpec((1, RB, 128), lambda i, j: (j, i, 0)),
            pl.BlockSpec((RB, H), lambda i, j: (i, 0)),
            pl.BlockSpec((RB, H), lambda i, j: (i, 0)),
        ],
        out_shape=[
            jax.ShapeDtypeStruct((nchunk, NP, 128), jnp.float32),
            jax.ShapeDtypeStruct((NP, H), jnp.float32),
            jax.ShapeDtypeStruct((NP, H), jnp.float32),
        ],
    )(z, W, As8, Ad8)


def _epi_concat_body(agg_ref, z_ref, wl_ref, b_ref, bl_ref, out_ref):
    skip = jnp.dot(z_ref[...], wl_ref[...], preferred_element_type=jnp.float32)
    v = agg_ref[0] + b_ref[0][None, :] + skip + bl_ref[0][None, :]
    out_ref[...] = jnp.where(v > 0, v, jnp.exp(v) - 1.0)


def tc_epilogue_concat(agg_t, z, Wl, b, bl):
    Din = z.shape[1]
    F = Wl.shape[1]
    nchunk = F // 128
    b2 = b.reshape(1, F)
    bl2 = bl.reshape(1, F)
    return pl.pallas_call(
        _epi_concat_body,
        grid=(NBLK, nchunk),
        in_specs=[
            pl.BlockSpec((1, RB, 128), lambda i, j: (j, i, 0)),
            pl.BlockSpec((RB, Din), lambda i, j: (i, 0)),
            pl.BlockSpec((Din, 128), lambda i, j: (0, j)),
            pl.BlockSpec((1, 128), lambda i, j: (0, j)),
            pl.BlockSpec((1, 128), lambda i, j: (0, j)),
        ],
        out_specs=pl.BlockSpec((RB, 128), lambda i, j: (i, j)),
        out_shape=jax.ShapeDtypeStruct((NP, F), jnp.float32),
    )(agg_t, z, Wl, b2, bl2)


def _epi_mean_body(agg_ref, z_ref, wl_ref, b_ref, bl_ref, out_ref):
    m = agg_ref[0]
    for k in range(1, 6):
        m = m + agg_ref[k]
    m = m * (1.0 / 6.0)
    skip = jnp.dot(z_ref[...], wl_ref[...], preferred_element_type=jnp.float32)
    out_ref[...] = m + b_ref[0][None, :] + skip + bl_ref[0][None, :]


def tc_epilogue_mean(agg_t, z, Wl, b, bl):
    Din = z.shape[1]
    b2 = b.reshape(1, 128)
    bl2 = bl.reshape(1, 128)
    return pl.pallas_call(
        _epi_mean_body,
        grid=(NBLK,),
        in_specs=[
            pl.BlockSpec((6, RB, 128), lambda i: (0, i, 0)),
            pl.BlockSpec((RB, Din), lambda i: (i, 0)),
            pl.BlockSpec((Din, 128), lambda i: (0, 0)),
            pl.BlockSpec((1, 128), lambda i: (0, 0)),
            pl.BlockSpec((1, 128), lambda i: (0, 0)),
        ],
        out_specs=pl.BlockSpec((RB, 128), lambda i: (i, 0)),
        out_shape=jax.ShapeDtypeStruct((NP, 128), jnp.float32),
    )(agg_t, z, Wl, b2, bl2)


_SC_MESH = plsc.VectorSubcoreMesh(core_axis_name="c", subcore_axis_name="s",
                                  num_cores=2, num_subcores=16)


def _zero_1d(ref, n):
    zeros = jnp.zeros((16,), jnp.float32)

    def body(i, _):
        ref[pl.ds(i * 16, 16)] = zeros
        return 0

    lax.fori_loop(0, n // 16, body, 0)


def make_sc_coef(H):
    Hc = H // 2
    KC = 512
    NB1 = EPT // KC

    scratch = []
    scratch += [pltpu.VMEM((NP,), jnp.float32) for _ in range(2 * Hc)]
    scratch += [pltpu.VMEM((NRSL,), jnp.float32) for _ in range(Hc)]
    scratch += [
        pltpu.VMEM((KC,), jnp.int32),
        pltpu.VMEM((KC,), jnp.int32),
        pltpu.VMEM((KC,), jnp.float32),
        pltpu.VMEM((ASTRIPE,), jnp.float32),
    ]
    scratch += [pltpu.VMEM((ASTRIPE,), jnp.float32) for _ in range(16)]
    scratch += [pltpu.SemaphoreType.DMA]

    @functools.partial(
        pl.kernel,
        out_type=(
            jax.ShapeDtypeStruct((H, EPAD), jnp.float32),
            jax.ShapeDtypeStruct((32 * Hc * NRSL,), jnp.float32),
            jax.ShapeDtypeStruct((H * NRSL,), jnp.float32),
        ),
        mesh=_SC_MESH,
        compiler_params=pltpu.CompilerParams(needs_layout_passes=False),
        scratch_types=scratch,
    )
    def sc_coef(asrcT, adstT, srcp, dstp, coef_out, part, sfin, *refs):
        as_pl = refs[0:Hc]
        ad_pl = refs[Hc:2 * Hc]
        s_pl = refs[2 * Hc:3 * Hc]
        sbuf, dbuf, cbuf, cacc = refs[3 * Hc:3 * Hc + 4]
        ctmp = refs[3 * Hc + 4:3 * Hc + 20]
        sem = refs[3 * Hc + 20]

        core = lax.axis_index("c")
        tile = lax.axis_index("s")
        base_e = tile * EPT
        soff = tile * ASTRIPE
        wid = core * 16 + tile

        for hh in range(Hc):
            pltpu.sync_copy(asrcT.at[core * Hc + hh], as_pl[hh])
            pltpu.sync_copy(adstT.at[core * Hc + hh], ad_pl[hh])
            _zero_1d(s_pl[hh], NRSL)

        def batch_a(b, _):
            off = base_e + b * KC
            pltpu.sync_copy(srcp.at[pl.ds(off, KC)], sbuf)
            pltpu.sync_copy(dstp.at[pl.ds(off, KC)], dbuf)

            def vec(v, _):
                sl = pl.ds(v * 16, 16)
                sv = sbuf[sl]
                dv = dbuf[sl]
                for hh in range(Hc):
                    av = (plsc.load_gather(as_pl[hh], [sv])
                          + plsc.load_gather(ad_pl[hh], [dv]))
                    av = jnp.where(av >= 0, av, 0.2 * av)
                    plsc.addupdate_scatter(s_pl[hh], [dv], jnp.exp(av))
                return 0

            lax.fori_loop(0, KC // 16, vec, 0)
            return 0

        lax.fori_loop(0, NB1, batch_a, 0)
        for hh in range(Hc):
            pltpu.sync_copy(s_pl[hh],
                            part.at[pl.ds((wid * Hc + hh) * NRSL, NRSL)])

        plsc.subcore_barrier()

        for hh in range(Hc):
            for p in range(16):
                pltpu.async_copy(
                    part.at[pl.ds(((core * 16 + p) * Hc + hh) * NRSL + soff,
                                  ASTRIPE)],
                    ctmp[p], sem)
            for p in range(16):
                pltpu.make_async_copy(
                    part.at[pl.ds(((core * 16 + p) * Hc + hh) * NRSL + soff,
                                  ASTRIPE)],
                    ctmp[p], sem).wait()
            _zero_1d(cacc, ASTRIPE)

            def vadd(v, _):
                sl = pl.ds(v * 16, 16)
                acc16 = cacc[sl]
                for p in range(16):
                    acc16 = acc16 + ctmp[p][sl]
                cacc[sl] = acc16
                return 0

            lax.fori_loop(0, ASTRIPE // 16, vadd, 0)
            pltpu.sync_copy(
                cacc, sfin.at[pl.ds((core * Hc + hh) * NRSL + soff, ASTRIPE)])

        plsc.subcore_barrier()

        for hh in range(Hc):
            pltpu.sync_copy(sfin.at[pl.ds((core * Hc + hh) * NRSL, NRSL)],
                            s_pl[hh])

        def batch_b(b, _):
            off = base_e + b * KC
            pltpu.sync_copy(srcp.at[pl.ds(off, KC)], sbuf)
            pltpu.sync_copy(dstp.at[pl.ds(off, KC)], dbuf)
            for hh in range(Hc):
                def vec(v, _):
                    sl = pl.ds(v * 16, 16)
                    sv = sbuf[sl]
                    dv = dbuf[sl]
                    av = (plsc.load_gather(as_pl[hh], [sv])
                          + plsc.load_gather(ad_pl[hh], [dv]))
                    av = jnp.where(av >= 0, av, 0.2 * av)
                    ev = jnp.exp(av)
                    sval = plsc.load_gather(s_pl[hh], [dv])
                    cbuf[sl] = ev / (sval + 1e-16)
                    return 0

                lax.fori_loop(0, KC // 16, vec, 0)
                pltpu.sync_copy(cbuf,
                                coef_out.at[core * Hc + hh, pl.ds(off, KC)])
            return 0

        lax.fori_loop(0, NB1, batch_b, 0)

    return sc_coef


NRACC = NRSL


def make_sc_aggregate(nchunk, H):
    CPC = nchunk // 2
    sub = nchunk // H
    KA = 128
    NBA = EPT // KA
    NB2 = NBA // 2

    idx_t = pltpu.VMEM((1, 128), jnp.int32)
    cf_t = pltpu.VMEM((1, 128), jnp.float32)

    @functools.partial(
        pl.kernel,
        out_type=jax.ShapeDtypeStruct((nchunk, NRACC, 128), jnp.float32),
        mesh=_SC_MESH,
        compiler_params=pltpu.CompilerParams(needs_layout_passes=False),
        scratch_types=[
            idx_t, idx_t, cf_t, idx_t, idx_t,
            idx_t, idx_t, cf_t, idx_t, idx_t,
            pltpu.VMEM((KA, 128), jnp.float32),
            pltpu.VMEM((KA, 128), jnp.float32),
            pltpu.SemaphoreType.DMA,
            pltpu.SemaphoreType.DMA,
            pltpu.SemaphoreType.DMA,
            pltpu.SemaphoreType.DMA,
            pltpu.SemaphoreType.DMA,
            pltpu.SemaphoreType.DMA,
            pltpu.VMEM_SHARED((NRACC, 128), jnp.float32),
        ],
    )
    def sc_agg(ht, coef, srcp, dstp, agg_out,
               sb0, db0, cb0, gb0, x0, sb1, db1, cb1, gb1, x1, rw0, rw1,
               semg0, semg1, sems0, sems1, semi0, semi1, acc):
        core = lax.axis_index("c")
        tile = lax.axis_index("s")
        base_e = tile * EPT
        soff = tile * ASTRIPE
        zeros = jnp.zeros((16,), jnp.float32)
        bufs = ((sb0, db0, cb0, gb0, x0, rw0, semg0, sems0, semi0),
                (sb1, db1, cb1, gb1, x1, rw1, semg1, sems1, semi1))

        def zero_stripe():
            def zrow(r, _):
                for j in range(8):
                    rw0[r, pl.ds(j * 16, 16)] = zeros
                return 0

            lax.fori_loop(0, KA, zrow, 0)
            for i in range(ASTRIPE // KA):
                pltpu.sync_copy(rw0, acc.at[pl.ds(soff + i * KA, KA)])
            rem = ASTRIPE % KA
            pltpu.sync_copy(rw0.at[pl.ds(0, rem)],
                            acc.at[pl.ds(soff + ASTRIPE - rem, rem)])

        zero_stripe()

        for cc in range(CPC):
            c = core * CPC + cc
            head = c // sub
            cbase = c * NP

            plsc.subcore_barrier()

            def start_idx(which, b):
                sb, db, cb = bufs[which][:3]
                semi = bufs[which][8]
                off = base_e + b * KA
                pltpu.async_copy(srcp.at[pl.ds(off, 128)], sb.at[0], semi)
                pltpu.async_copy(dstp.at[pl.ds(off, 128)], db.at[0], semi)
                pltpu.async_copy(coef.at[head, pl.ds(off, 128)], cb.at[0],
                                 semi)

            def wait_idx(which, b):
                sb, db, cb = bufs[which][:3]
                semi = bufs[which][8]
                off = base_e + b * KA
                pltpu.make_async_copy(srcp.at[pl.ds(off, 128)], sb.at[0],
                                      semi).wait()
                pltpu.make_async_copy(dstp.at[pl.ds(off, 128)], db.at[0],
                                      semi).wait()
                pltpu.make_async_copy(coef.at[head, pl.ds(off, 128)],
                                      cb.at[0], semi).wait()

            def gidx(which):
                sb, gb = bufs[which][0], bufs[which][3]

                def gv(v, _):
                    sl = pl.ds(v * 16, 16)
                    gb[0, sl] = sb[0, sl] + cbase
                    return 0

                lax.fori_loop(0, 8, gv, 0)

            def start_g(which):
                gb, rw, semg = bufs[which][3], bufs[which][5], bufs[which][6]
                pltpu.async_copy(ht.at[gb.at[0]], rw, semg)

            def wait_g(which):
                gb, rw, semg = bufs[which][3], bufs[which][5], bufs[which][6]
                pltpu.make_async_copy(ht.at[gb.at[0]], rw, semg).wait()

            def start_s(which):
                xb, rw, sems = bufs[which][4], bufs[which][5], bufs[which][7]
                pltpu.async_copy(rw, acc.at[xb.at[0]], sems, add=True)

            def wait_s(which):
                xb, rw, sems = bufs[which][4], bufs[which][5], bufs[which][7]
                pltpu.make_async_copy(rw, acc.at[xb.at[0]], sems).wait()

            def process(which):
                db, cb, xb, rw = (bufs[which][1], bufs[which][2],
                                  bufs[which][4], bufs[which][5])

                def mul(v, _):
                    cv = cb[0, pl.ds(v * 16, 16)]
                    for lane in range(16):
                        e = v * 16 + lane
                        cf = cv[lane]
                        for q in range(8):
                            sl = pl.ds(q * 16, 16)
                            rw[e, sl] = rw[e, sl] * cf
                    return 0

                lax.fori_loop(0, 8, mul, 0)

                def cpy(v, _):
                    sl = pl.ds(v * 16, 16)
                    xb[0, sl] = db[0, sl]
                    return 0

                lax.fori_loop(0, 8, cpy, 0)
                start_s(which)

            start_idx(0, 0)
            wait_idx(0, 0)
            gidx(0)
            start_g(0)
            start_idx(1, 1)

            def pair(i, _):
                wait_idx(1, 2 * i + 1)
                gidx(1)

                @pl.when(i > 0)
                def _():
                    wait_s(1)

                start_g(1)
                wait_g(0)
                process(0)

                @pl.when(i < NB2 - 1)
                def _():
                    start_idx(0, 2 * i + 2)

                wait_g(1)
                process(1)

                @pl.when(i < NB2 - 1)
                def _():
                    wait_idx(0, 2 * i + 2)
                    gidx(0)
                    wait_s(0)
                    start_g(0)
                    start_idx(1, 2 * i + 3)
                return 0

            lax.fori_loop(0, NB2, pair, 0)
            wait_s(0)
            wait_s(1)

            plsc.subcore_barrier()

            pltpu.sync_copy(acc.at[pl.ds(soff, ASTRIPE)],
                            agg_out.at[c, pl.ds(soff, ASTRIPE)])
            zero_stripe()

    return sc_agg


def _att_blockdiag(att, F):
    H, C = att.shape
    nchunk = F // 128
    sub = C // 128
    A = jnp.zeros((nchunk, 128, H), jnp.float32)
    for c in range(nchunk):
        hd = c // sub
        block = att[hd, (c % sub) * 128:(c % sub + 1) * 128]
        A = A.at[c, :, hd].set(block)
    return A


def _gat_layer(z, src, dst, W, att_s, att_d, b, Wl, bl, heads, concat):
    F = W.shape[1]
    nchunk = F // 128
    As8 = _att_blockdiag(att_s, F)
    Ad8 = _att_blockdiag(att_d, F)
    h_t, asrc, adst = tc_prologue(z, W, As8, Ad8)
    coef_p, _, _ = make_sc_coef(heads)(asrc.T, adst.T, src, dst)
    ht_flat = h_t.reshape(nchunk * NP, 128)
    agg_t = make_sc_aggregate(nchunk, heads)(ht_flat, coef_p, src, dst)
    if concat:
        return tc_epilogue_concat(agg_t, z, Wl, b, bl)
    return tc_epilogue_mean(agg_t, z, Wl, b, bl)


def kernel(x, edge_index, W1, as1, ad1, b1, Wl1, bl1, W2, as2, ad2, b2, Wl2,
           bl2, W3, as3, ad3, b3, Wl3, bl3):
    loop = jnp.arange(N, dtype=edge_index.dtype)
    ei = jnp.concatenate([edge_index, jnp.stack([loop, loop])], axis=1)
    src = jnp.full((EPAD,), N, jnp.int32).at[:EDGES].set(ei[0].astype(jnp.int32))
    dst = jnp.full((EPAD,), N, jnp.int32).at[:EDGES].set(ei[1].astype(jnp.int32))

    xp = jnp.zeros((NP, x.shape[1]), jnp.float32).at[:N].set(x)
    h = _gat_layer(xp, src, dst, W1, as1, ad1, b1, Wl1, bl1, 4, True)
    h = _gat_layer(h, src, dst, W2, as2, ad2, b2, Wl2, bl2, 4, True)
    out = _gat_layer(h, src, dst, W3, as3, ad3, b3, Wl3, bl3, 6, False)
    return out[:N]
